# baseline (speedup 1.0000x reference)
import jax
import jax.numpy as jnp
from jax import lax
from jax.experimental import pallas as pl
from jax.experimental.pallas import tpu as pltpu
from jax.experimental.pallas import tpu_sc as plsc

_DIM = 128
_BUCKET = 64
_LANES = 16
_NEG = -3.4028234663852886e38


def _sc_body(k_hbm, w_hbm, bk_hbm, ws_hbm, wbuf, kb0, kb1, obk, ows,
             sem0, sem1):
    bh = k_hbm.shape[0]
    ngrp = _DIM // _LANES
    wid = lax.axis_index("s") * 2 + lax.axis_index("c")

    pltpu.sync_copy(w_hbm.at[pl.ds(2 * wid, 2)], wbuf)

    zero = jnp.zeros((_LANES,), jnp.float32)

    def accumulate(kb, jj, bh_idx):
        def s_body(s, accs):
            wv = wbuf[jj, s]
            new = []
            for g in range(ngrp):
                row = kb[s, pl.ds(_LANES * g, _LANES)]
                new.append(accs[g] + row)
                new.append(accs[ngrp + g] + wv * row)
            return tuple(new[0::2]) + tuple(new[1::2])

        accs = lax.fori_loop(0, _BUCKET, s_body, (zero,) * (2 * ngrp))
        for g in range(ngrp):
            obk[bh_idx, pl.ds(_LANES * g, _LANES)] = accs[g]
            ows[bh_idx, pl.ds(_LANES * g, _LANES)] = accs[ngrp + g]

    for jj in range(2):
        j = 2 * wid + jj
        pltpu.async_copy(k_hbm.at[0, j], kb0, sem0)
        pltpu.async_copy(k_hbm.at[1, j], kb1, sem1)

        def t_body(t, carry):
            b0 = 2 * t
            b1 = 2 * t + 1
            pltpu.make_async_copy(k_hbm.at[b0, j], kb0, sem0).wait()
            accumulate(kb0, jj, b0)

            @pl.when(t < bh // 2 - 1)
            def _():
                pltpu.async_copy(k_hbm.at[b0 + 2, j], kb0, sem0)

            pltpu.make_async_copy(k_hbm.at[b1, j], kb1, sem1).wait()
            accumulate(kb1, jj, b1)

            @pl.when(t < bh // 2 - 1)
            def _():
                pltpu.async_copy(k_hbm.at[b1 + 2, j], kb1, sem1)

            return carry

        lax.fori_loop(0, bh // 2, t_body, 0)

        pltpu.sync_copy(obk, bk_hbm.at[j])
        pltpu.sync_copy(ows, ws_hbm.at[j])


def _k_side_sc(k4, wtab):
    bh, nb = k4.shape[0], k4.shape[1]
    out_t = jax.ShapeDtypeStruct((nb, bh, _DIM), jnp.float32)
    f = pl.kernel(
        _sc_body,
        out_type=(out_t, out_t),
        mesh=plsc.VectorSubcoreMesh(core_axis_name="c", subcore_axis_name="s",
                                    num_cores=2, num_subcores=16),
        scratch_types=[
            pltpu.VMEM((2, _BUCKET, _LANES), jnp.float32),
            pltpu.VMEM((_BUCKET, _DIM), jnp.float32),
            pltpu.VMEM((_BUCKET, _DIM), jnp.float32),
            pltpu.VMEM((bh, _DIM), jnp.float32),
            pltpu.VMEM((bh, _DIM), jnp.float32),
            pltpu.SemaphoreType.DMA,
            pltpu.SemaphoreType.DMA,
        ],
    )
    return f(k4, wtab)


def _tail_body(scale_ref, q_ref, bk_ref, ws_ref, o_ref):
    q3 = q_ref[0]
    bk = bk_ref[0]
    ws = ws_ref[0]
    nb = q3.shape[0]
    f32 = jnp.float32

    r64 = lax.broadcasted_iota(jnp.int32, (nb, _BUCKET), 0)
    c64 = lax.broadcasted_iota(jnp.int32, (nb, _BUCKET), 1)

    rinv = 1.0 / (_BUCKET * r64 + c64 + 1).astype(f32)
    h = jnp.sum(rinv, axis=1, keepdims=True)

    bq = jnp.sum(q3, axis=1)
    l_strict = (r64 > c64).astype(f32)
    pq = jnp.dot(l_strict, bq, preferred_element_type=f32,
                 precision=lax.Precision.HIGHEST)
    pk = jnp.dot(l_strict, bk, preferred_element_type=f32,
                 precision=lax.Precision.HIGHEST)

    sk = pk * h + ws
    inv_cnt = 1.0 / (_BUCKET * r64[:, :1] + 1).astype(f32)
    sq = (pq + q3[:, 0, :]) * inv_cnt

    scale = scale_ref[0]
    r_core = lax.dot_general(sq, sk, (((1,), (1,)), ((), ())),
                             preferred_element_type=f32) * scale
    r_core = jnp.where(c64 >= r64, _NEG, r_core)

    m = jnp.maximum(jnp.max(r_core, axis=1, keepdims=True), 0.0)
    e = jnp.exp(r_core - m)
    p0 = jnp.exp(-m)
    s = p0 + jnp.sum(e, axis=1, keepdims=True)
    p_core = e / s
    p0 = p0 / s
    v = jnp.maximum(jnp.max(p_core, axis=1, keepdims=True), p0)
    cand = jnp.where(p_core == v, c64 + 1, 2 * _BUCKET)
    amin = jnp.min(cand, axis=1, keepdims=True)
    amin = jnp.where(p0 == v, 0, amin)

    ccol = lax.broadcasted_iota(jnp.int32, (nb, _DIM), 1)
    o_ref[0] = jnp.where(ccol == amin, v, 0.0)


def kernel(q, k, topk):
    bh, seq, dim = q.shape
    nb = seq // _BUCKET
    q4 = q.reshape(bh, nb, _BUCKET, dim)
    k4 = k.reshape(bh, nb, _BUCKET, dim)
    scale = (jnp.asarray(topk, jnp.float32) * (dim ** -0.5)).reshape(1)

    pos = jnp.arange(seq, dtype=jnp.float32).reshape(nb, _BUCKET)
    rinv = 1.0 / (pos + 1.0)
    w = jnp.cumsum(rinv[:, ::-1], axis=1)[:, ::-1]
    wtab = jnp.broadcast_to(w[:, :, None], (nb, _BUCKET, _LANES))

    bk_t, ws_t = _k_side_sc(k4, wtab)
    bk = jnp.transpose(bk_t, (1, 0, 2))
    ws = jnp.transpose(ws_t, (1, 0, 2))

    out = pl.pallas_call(
        _tail_body,
        grid=(bh,),
        in_specs=[
            pl.BlockSpec(memory_space=pltpu.SMEM),
            pl.BlockSpec((1, nb, _BUCKET, dim), lambda b: (b, 0, 0, 0)),
            pl.BlockSpec((1, nb, _DIM), lambda b: (b, 0, 0)),
            pl.BlockSpec((1, nb, _DIM), lambda b: (b, 0, 0)),
        ],
        out_specs=pl.BlockSpec((1, nb, _DIM), lambda b: (b, 0, 0)),
        out_shape=jax.ShapeDtypeStruct((bh, nb, _DIM), jnp.float32),
        compiler_params=pltpu.CompilerParams(
            dimension_semantics=("arbitrary",),
        ),
    )(scale, q4, bk, ws)
    return out[:, :, : nb + 1]


# speedup vs baseline: 1.0591x; 1.0591x over previous
import jax
import jax.numpy as jnp
from jax import lax
from jax.experimental import pallas as pl
from jax.experimental.pallas import tpu as pltpu
from jax.experimental.pallas import tpu_sc as plsc

_DIM = 128
_BUCKET = 64
_LANES = 16
_NEG = -3.4028234663852886e38


def _sc_body(k_hbm, w_hbm, bk_hbm, ws_hbm, wbuf, kb0, kb1, obk, ows,
             sem0, sem1):
    bh = k_hbm.shape[0]
    ngrp = _DIM // _LANES
    wid = lax.axis_index("s") * 2 + lax.axis_index("c")

    pltpu.sync_copy(w_hbm.at[pl.ds(2 * wid, 2)], wbuf)

    zero = jnp.zeros((_LANES,), jnp.float32)

    def accumulate(kb, jj, bh_idx):
        def s_body(s, accs):
            wv = wbuf[jj, s]
            new = []
            for g in range(ngrp):
                row = kb[s, pl.ds(_LANES * g, _LANES)]
                new.append(accs[g] + row)
                new.append(accs[ngrp + g] + wv * row)
            return tuple(new[0::2]) + tuple(new[1::2])

        accs = lax.fori_loop(0, _BUCKET, s_body, (zero,) * (2 * ngrp))
        for g in range(ngrp):
            obk[bh_idx, pl.ds(_LANES * g, _LANES)] = accs[g]
            ows[bh_idx, pl.ds(_LANES * g, _LANES)] = accs[ngrp + g]

    for jj in range(2):
        j = 2 * wid + jj
        pltpu.async_copy(k_hbm.at[0, j], kb0, sem0)
        pltpu.async_copy(k_hbm.at[1, j], kb1, sem1)

        def t_body(t, carry):
            b0 = 2 * t
            b1 = 2 * t + 1
            pltpu.make_async_copy(k_hbm.at[b0, j], kb0, sem0).wait()
            accumulate(kb0, jj, b0)

            @pl.when(t < bh // 2 - 1)
            def _():
                pltpu.async_copy(k_hbm.at[b0 + 2, j], kb0, sem0)

            pltpu.make_async_copy(k_hbm.at[b1, j], kb1, sem1).wait()
            accumulate(kb1, jj, b1)

            @pl.when(t < bh // 2 - 1)
            def _():
                pltpu.async_copy(k_hbm.at[b1 + 2, j], kb1, sem1)

            return carry

        lax.fori_loop(0, bh // 2, t_body, 0)

        pltpu.sync_copy(obk, bk_hbm.at[j])
        pltpu.sync_copy(ows, ws_hbm.at[j])


def _k_side_sc(k4, wtab):
    bh, nb = k4.shape[0], k4.shape[1]
    out_t = jax.ShapeDtypeStruct((nb, bh, _DIM), jnp.float32)
    f = pl.kernel(
        _sc_body,
        out_type=(out_t, out_t),
        mesh=plsc.VectorSubcoreMesh(core_axis_name="c", subcore_axis_name="s",
                                    num_cores=2, num_subcores=16),
        scratch_types=[
            pltpu.VMEM((2, _BUCKET, _LANES), jnp.float32),
            pltpu.VMEM((_BUCKET, _DIM), jnp.float32),
            pltpu.VMEM((_BUCKET, _DIM), jnp.float32),
            pltpu.VMEM((bh, _DIM), jnp.float32),
            pltpu.VMEM((bh, _DIM), jnp.float32),
            pltpu.SemaphoreType.DMA,
            pltpu.SemaphoreType.DMA,
        ],
    )
    return f(k4, wtab)


def _qside_body(scale_ref, q_ref, sq_ref):
    q3 = q_ref[0]
    nb = q3.shape[0]
    f32 = jnp.float32

    r64 = lax.broadcasted_iota(jnp.int32, (nb, _BUCKET), 0)
    c64 = lax.broadcasted_iota(jnp.int32, (nb, _BUCKET), 1)

    bq = jnp.sum(q3, axis=1)
    l_strict = (r64 > c64).astype(f32)
    pq = jnp.dot(l_strict, bq, preferred_element_type=f32,
                 precision=lax.Precision.HIGHEST)
    inv_cnt = 1.0 / (_BUCKET * r64[:, :1] + 1).astype(f32)
    sq_ref[0] = (pq + q3[:, 0, :]) * (inv_cnt * scale_ref[0])


def _tail_body(sq_ref, bk_ref, ws_ref, o_ref):
    sq = sq_ref[0]
    bk = bk_ref[...]
    ws = ws_ref[...]
    nb = sq.shape[0]
    f32 = jnp.float32

    r64 = lax.broadcasted_iota(jnp.int32, (nb, _BUCKET), 0)
    c64 = lax.broadcasted_iota(jnp.int32, (nb, _BUCKET), 1)

    rinv = 1.0 / (_BUCKET * r64 + c64 + 1).astype(f32)
    h = jnp.sum(rinv, axis=1, keepdims=True)

    l_strict = (r64 > c64).astype(f32)
    pk = jnp.dot(l_strict, bk, preferred_element_type=f32,
                 precision=lax.Precision.HIGHEST)
    sk = pk * h + ws

    r_core = lax.dot_general(sq, sk, (((1,), (1,)), ((), ())),
                             preferred_element_type=f32)
    r_core = jnp.where(c64 >= r64, _NEG, r_core)

    m = jnp.maximum(jnp.max(r_core, axis=1, keepdims=True), 0.0)
    e = jnp.exp(r_core - m)
    p0 = jnp.exp(-m)
    s = p0 + jnp.sum(e, axis=1, keepdims=True)
    p_core = e / s
    p0 = p0 / s
    v = jnp.maximum(jnp.max(p_core, axis=1, keepdims=True), p0)
    cand = jnp.where(p_core == v, c64 + 1, 2 * _BUCKET)
    amin = jnp.min(cand, axis=1, keepdims=True)
    amin = jnp.where(p0 == v, 0, amin)

    ccol = lax.broadcasted_iota(jnp.int32, (nb, _DIM), 1)
    o_ref[0] = jnp.where(ccol == amin, v, 0.0)


def kernel(q, k, topk):
    bh, seq, dim = q.shape
    nb = seq // _BUCKET
    q4 = q.reshape(bh, nb, _BUCKET, dim)
    k4 = k.reshape(bh, nb, _BUCKET, dim)
    scale = (jnp.asarray(topk, jnp.float32) * (dim ** -0.5)).reshape(1)

    pos = jnp.arange(seq, dtype=jnp.float32).reshape(nb, _BUCKET)
    rinv = 1.0 / (pos + 1.0)
    w = jnp.cumsum(rinv[:, ::-1], axis=1)[:, ::-1]
    wtab = jnp.broadcast_to(w[:, :, None], (nb, _BUCKET, _LANES))

    bk_t, ws_t = _k_side_sc(k4, wtab)
    bk2 = bk_t.reshape(nb, bh * _DIM)
    ws2 = ws_t.reshape(nb, bh * _DIM)

    sqs = pl.pallas_call(
        _qside_body,
        grid=(bh,),
        in_specs=[
            pl.BlockSpec(memory_space=pltpu.SMEM),
            pl.BlockSpec((1, nb, _BUCKET, dim), lambda b: (b, 0, 0, 0)),
        ],
        out_specs=pl.BlockSpec((1, nb, _DIM), lambda b: (b, 0, 0)),
        out_shape=jax.ShapeDtypeStruct((bh, nb, _DIM), jnp.float32),
        compiler_params=pltpu.CompilerParams(
            dimension_semantics=("arbitrary",),
        ),
    )(scale, q4)

    out = pl.pallas_call(
        _tail_body,
        grid=(bh,),
        in_specs=[
            pl.BlockSpec((1, nb, _DIM), lambda b: (b, 0, 0)),
            pl.BlockSpec((nb, _DIM), lambda b: (0, b)),
            pl.BlockSpec((nb, _DIM), lambda b: (0, b)),
        ],
        out_specs=pl.BlockSpec((1, nb, _DIM), lambda b: (b, 0, 0)),
        out_shape=jax.ShapeDtypeStruct((bh, nb, _DIM), jnp.float32),
        compiler_params=pltpu.CompilerParams(
            dimension_semantics=("arbitrary",),
        ),
    )(sqs, bk2, ws2)
    return out[:, :, : nb + 1]


# speedup vs baseline: 1.0604x; 1.0012x over previous
import jax
import jax.numpy as jnp
from jax import lax
from jax.experimental import pallas as pl
from jax.experimental.pallas import tpu as pltpu
from jax.experimental.pallas import tpu_sc as plsc

_DIM = 128
_BUCKET = 64
_LANES = 16
_NEG = -3.4028234663852886e38


def _sc_body(k_hbm, w_hbm, bk_hbm, ws_hbm, wbuf, kb0, kb1, obk, ows,
             sem0, sem1):
    bh = k_hbm.shape[0]
    ngrp = _DIM // _LANES
    wid = lax.axis_index("s") * 2 + lax.axis_index("c")

    pltpu.sync_copy(w_hbm.at[pl.ds(2 * wid, 2)], wbuf)

    zero = jnp.zeros((_LANES,), jnp.float32)

    def accumulate(kb, jj, bh_idx):
        def s_body(s, accs):
            wv = wbuf[jj, s]
            new = []
            for g in range(ngrp):
                row = kb[s, pl.ds(_LANES * g, _LANES)]
                new.append(accs[g] + row)
                new.append(accs[ngrp + g] + wv * row)
            return tuple(new[0::2]) + tuple(new[1::2])

        accs = lax.fori_loop(0, _BUCKET, s_body, (zero,) * (2 * ngrp))
        for g in range(ngrp):
            obk[bh_idx, pl.ds(_LANES * g, _LANES)] = accs[g]
            ows[bh_idx, pl.ds(_LANES * g, _LANES)] = accs[ngrp + g]

    for jj in range(2):
        j = 2 * wid + jj
        pltpu.async_copy(k_hbm.at[0, j], kb0, sem0)
        pltpu.async_copy(k_hbm.at[1, j], kb1, sem1)

        def t_body(t, carry):
            b0 = 2 * t
            b1 = 2 * t + 1
            pltpu.make_async_copy(k_hbm.at[b0, j], kb0, sem0).wait()
            accumulate(kb0, jj, b0)

            @pl.when(t < bh // 2 - 1)
            def _():
                pltpu.async_copy(k_hbm.at[b0 + 2, j], kb0, sem0)

            pltpu.make_async_copy(k_hbm.at[b1, j], kb1, sem1).wait()
            accumulate(kb1, jj, b1)

            @pl.when(t < bh // 2 - 1)
            def _():
                pltpu.async_copy(k_hbm.at[b1 + 2, j], kb1, sem1)

            return carry

        lax.fori_loop(0, bh // 2, t_body, 0)

        pltpu.sync_copy(obk, bk_hbm.at[j])
        pltpu.sync_copy(ows, ws_hbm.at[j])


def _k_side_sc(k4, wtab):
    bh, nb = k4.shape[0], k4.shape[1]
    out_t = jax.ShapeDtypeStruct((nb, bh, _DIM), jnp.float32)
    f = pl.kernel(
        _sc_body,
        out_type=(out_t, out_t),
        mesh=plsc.VectorSubcoreMesh(core_axis_name="c", subcore_axis_name="s",
                                    num_cores=2, num_subcores=16),
        scratch_types=[
            pltpu.VMEM((2, _BUCKET, _LANES), jnp.float32),
            pltpu.VMEM((_BUCKET, _DIM), jnp.float32),
            pltpu.VMEM((_BUCKET, _DIM), jnp.float32),
            pltpu.VMEM((bh, _DIM), jnp.float32),
            pltpu.VMEM((bh, _DIM), jnp.float32),
            pltpu.SemaphoreType.DMA,
            pltpu.SemaphoreType.DMA,
        ],
    )
    return f(k4, wtab)


def _qside_body(q_ref, sq_ref):
    q3 = q_ref[0]
    nb = q3.shape[0]
    f32 = jnp.float32

    r64 = lax.broadcasted_iota(jnp.int32, (nb, _BUCKET), 0)
    c64 = lax.broadcasted_iota(jnp.int32, (nb, _BUCKET), 1)

    bq = jnp.sum(q3, axis=1)
    l_strict = (r64 > c64).astype(f32)
    pq = jnp.dot(l_strict, bq, preferred_element_type=f32,
                 precision=lax.Precision.HIGHEST)
    inv_cnt = 1.0 / (_BUCKET * r64[:, :1] + 1).astype(f32)
    sq_ref[0] = (pq + q3[:, 0, :]) * inv_cnt


def _tail_body(scale_ref, sq_ref, bk_ref, ws_ref, o_ref):
    sq = sq_ref[0]
    bk = bk_ref[...]
    ws = ws_ref[...]
    nb = sq.shape[0]
    f32 = jnp.float32

    r64 = lax.broadcasted_iota(jnp.int32, (nb, _BUCKET), 0)
    c64 = lax.broadcasted_iota(jnp.int32, (nb, _BUCKET), 1)

    rinv = 1.0 / (_BUCKET * r64 + c64 + 1).astype(f32)
    h = jnp.sum(rinv, axis=1, keepdims=True)

    l_strict = (r64 > c64).astype(f32)
    pk = jnp.dot(l_strict, bk, preferred_element_type=f32,
                 precision=lax.Precision.HIGHEST)
    sk = pk * h + ws

    r_core = lax.dot_general(sq, sk, (((1,), (1,)), ((), ())),
                             preferred_element_type=f32) * scale_ref[0]
    r_core = jnp.where(c64 >= r64, _NEG, r_core)

    m = jnp.maximum(jnp.max(r_core, axis=1, keepdims=True), 0.0)
    e = jnp.exp(r_core - m)
    p0 = jnp.exp(-m)
    s = p0 + jnp.sum(e, axis=1, keepdims=True)
    p_core = e / s
    p0 = p0 / s
    v = jnp.maximum(jnp.max(p_core, axis=1, keepdims=True), p0)
    cand = jnp.where(p_core == v, c64 + 1, 2 * _BUCKET)
    amin = jnp.min(cand, axis=1, keepdims=True)
    amin = jnp.where(p0 == v, 0, amin)

    ccol = lax.broadcasted_iota(jnp.int32, (nb, _DIM), 1)
    o_ref[0] = jnp.where(ccol == amin, v, 0.0)


def kernel(q, k, topk):
    bh, seq, dim = q.shape
    nb = seq // _BUCKET
    q4 = q.reshape(bh, nb, _BUCKET, dim)
    k4 = k.reshape(bh, nb, _BUCKET, dim)
    scale = (jnp.asarray(topk, jnp.float32) * (dim ** -0.5)).reshape(1)

    pos = jnp.arange(seq, dtype=jnp.float32).reshape(nb, _BUCKET)
    rinv = 1.0 / (pos + 1.0)
    w = jnp.cumsum(rinv[:, ::-1], axis=1)[:, ::-1]
    wtab = jnp.broadcast_to(w[:, :, None], (nb, _BUCKET, _LANES))

    bk_t, ws_t = _k_side_sc(k4, wtab)
    bk2 = bk_t.reshape(nb, bh * _DIM)
    ws2 = ws_t.reshape(nb, bh * _DIM)

    sqs = pl.pallas_call(
        _qside_body,
        grid=(bh,),
        in_specs=[
            pl.BlockSpec((1, nb, _BUCKET, dim), lambda b: (b, 0, 0, 0)),
        ],
        out_specs=pl.BlockSpec((1, nb, _DIM), lambda b: (b, 0, 0)),
        out_shape=jax.ShapeDtypeStruct((bh, nb, _DIM), jnp.float32),
        compiler_params=pltpu.CompilerParams(
            dimension_semantics=("arbitrary",),
        ),
    )(q4)

    out = pl.pallas_call(
        _tail_body,
        grid=(bh,),
        in_specs=[
            pl.BlockSpec(memory_space=pltpu.SMEM),
            pl.BlockSpec((1, nb, _DIM), lambda b: (b, 0, 0)),
            pl.BlockSpec((nb, _DIM), lambda b: (0, b)),
            pl.BlockSpec((nb, _DIM), lambda b: (0, b)),
        ],
        out_specs=pl.BlockSpec((1, nb, _DIM), lambda b: (b, 0, 0)),
        out_shape=jax.ShapeDtypeStruct((bh, nb, _DIM), jnp.float32),
        compiler_params=pltpu.CompilerParams(
            dimension_semantics=("arbitrary",),
        ),
    )(scale, sqs, bk2, ws2)
    return out[:, :, : nb + 1]


# speedup vs baseline: 1.5304x; 1.4432x over previous
import jax
import jax.numpy as jnp
from jax import lax
from jax.experimental import pallas as pl
from jax.experimental.pallas import tpu as pltpu
from jax.experimental.pallas import tpu_sc as plsc

_DIM = 128
_BUCKET = 64
_LANES = 16
_NBUF = 4
_NEG = -3.4028234663852886e38


def _sc_body(k_hbm, w_hbm, bk_hbm, ws_hbm, wbuf, kb0, kb1, kb2, kb3,
             obk, ows, sem0, sem1, sem2, sem3):
    bh = k_hbm.shape[0]
    ngrp = _DIM // _LANES
    wid = lax.axis_index("s") * 2 + lax.axis_index("c")
    pd = pl.ds(2 * wid, 2)

    pltpu.sync_copy(w_hbm.at[pd], wbuf)

    bufs = (kb0, kb1, kb2, kb3)
    sems = (sem0, sem1, sem2, sem3)
    zero = jnp.zeros((_LANES,), jnp.float32)

    def accumulate(kb, bh_idx):
        for jj in range(2):
            def s_body(u, accs):
                new = list(accs)
                for v in range(2):
                    s = 2 * u + v
                    wv = wbuf[jj, s]
                    for g in range(ngrp):
                        row = kb[jj, s, pl.ds(_LANES * g, _LANES)]
                        new[g] = new[g] + row
                        new[ngrp + g] = new[ngrp + g] + wv * row
                return tuple(new)

            accs = lax.fori_loop(0, _BUCKET // 2, s_body, (zero,) * (2 * ngrp))
            for g in range(ngrp):
                obk[jj, bh_idx, pl.ds(_LANES * g, _LANES)] = accs[g]
                ows[jj, bh_idx, pl.ds(_LANES * g, _LANES)] = accs[ngrp + g]

    for c in range(_NBUF):
        pltpu.async_copy(k_hbm.at[c, pd], bufs[c], sems[c])

    def t_body(t, carry):
        for b in range(_NBUF):
            c = _NBUF * t + b
            pltpu.make_async_copy(k_hbm.at[c, pd], bufs[b], sems[b]).wait()
            accumulate(bufs[b], c)

            @pl.when(c + _NBUF < bh)
            def _():
                pltpu.async_copy(k_hbm.at[c + _NBUF, pd], bufs[b], sems[b])

        return carry

    lax.fori_loop(0, bh // _NBUF, t_body, 0)

    pltpu.sync_copy(obk, bk_hbm.at[pd])
    pltpu.sync_copy(ows, ws_hbm.at[pd])


def _k_side_sc(k4, wtab):
    bh, nb = k4.shape[0], k4.shape[1]
    out_t = jax.ShapeDtypeStruct((nb, bh, _DIM), jnp.float32)
    kbuf = pltpu.VMEM((2, _BUCKET, _DIM), jnp.float32)
    f = pl.kernel(
        _sc_body,
        out_type=(out_t, out_t),
        mesh=plsc.VectorSubcoreMesh(core_axis_name="c", subcore_axis_name="s",
                                    num_cores=2, num_subcores=16),
        scratch_types=[
            pltpu.VMEM((2, _BUCKET, _LANES), jnp.float32),
            kbuf, kbuf, kbuf, kbuf,
            pltpu.VMEM((2, bh, _DIM), jnp.float32),
            pltpu.VMEM((2, bh, _DIM), jnp.float32),
            pltpu.SemaphoreType.DMA,
            pltpu.SemaphoreType.DMA,
            pltpu.SemaphoreType.DMA,
            pltpu.SemaphoreType.DMA,
        ],
    )
    return f(k4, wtab)


def _qside_body(q_ref, sq_ref):
    q3 = q_ref[0]
    nb = q3.shape[0]
    f32 = jnp.float32

    r64 = lax.broadcasted_iota(jnp.int32, (nb, _BUCKET), 0)
    c64 = lax.broadcasted_iota(jnp.int32, (nb, _BUCKET), 1)

    bq = jnp.sum(q3, axis=1)
    l_strict = (r64 > c64).astype(f32)
    pq = jnp.dot(l_strict, bq, preferred_element_type=f32,
                 precision=lax.Precision.HIGHEST)
    inv_cnt = 1.0 / (_BUCKET * r64[:, :1] + 1).astype(f32)
    sq_ref[0] = (pq + q3[:, 0, :]) * inv_cnt


def _tail_body(scale_ref, sq_ref, bk_ref, ws_ref, o_ref):
    bh = sq_ref.shape[0]
    nb = sq_ref.shape[1]
    f32 = jnp.float32
    scale = scale_ref[0]

    r64 = lax.broadcasted_iota(jnp.int32, (nb, _BUCKET), 0)
    c64 = lax.broadcasted_iota(jnp.int32, (nb, _BUCKET), 1)

    rinv = 1.0 / (_BUCKET * r64 + c64 + 1).astype(f32)
    h = jnp.sum(rinv, axis=1, keepdims=True)

    l_strict = (r64 > c64).astype(f32)
    pk2 = jnp.dot(l_strict, bk_ref[...], preferred_element_type=f32,
                  precision=lax.Precision.HIGHEST)
    sk2 = pk2 * h + ws_ref[...]

    ccol = lax.broadcasted_iota(jnp.int32, (nb, _DIM), 1)
    for b in range(bh):
        sq = sq_ref[b]
        sk = sk2[:, b * _DIM:(b + 1) * _DIM]
        r_core = lax.dot_general(sq, sk, (((1,), (1,)), ((), ())),
                                 preferred_element_type=f32) * scale
        r_core = jnp.where(c64 >= r64, _NEG, r_core)

        m = jnp.maximum(jnp.max(r_core, axis=1, keepdims=True), 0.0)
        e = jnp.exp(r_core - m)
        p0 = jnp.exp(-m)
        s = p0 + jnp.sum(e, axis=1, keepdims=True)
        p_core = e / s
        p0 = p0 / s
        v = jnp.maximum(jnp.max(p_core, axis=1, keepdims=True), p0)
        cand = jnp.where(p_core == v, c64 + 1, 2 * _BUCKET)
        amin = jnp.min(cand, axis=1, keepdims=True)
        amin = jnp.where(p0 == v, 0, amin)

        o_ref[b] = jnp.where(ccol == amin, v, 0.0)


def kernel(q, k, topk):
    bh, seq, dim = q.shape
    nb = seq // _BUCKET
    q4 = q.reshape(bh, nb, _BUCKET, dim)
    k4 = k.reshape(bh, nb, _BUCKET, dim)
    scale = (jnp.asarray(topk, jnp.float32) * (dim ** -0.5)).reshape(1)

    pos = jnp.arange(seq, dtype=jnp.float32).reshape(nb, _BUCKET)
    rinv = 1.0 / (pos + 1.0)
    w = jnp.cumsum(rinv[:, ::-1], axis=1)[:, ::-1]
    wtab = jnp.broadcast_to(w[:, :, None], (nb, _BUCKET, _LANES))

    bk_t, ws_t = _k_side_sc(k4, wtab)
    bk2 = bk_t.reshape(nb, bh * _DIM)
    ws2 = ws_t.reshape(nb, bh * _DIM)

    sqs = pl.pallas_call(
        _qside_body,
        grid=(bh,),
        in_specs=[
            pl.BlockSpec((1, nb, _BUCKET, dim), lambda b: (b, 0, 0, 0)),
        ],
        out_specs=pl.BlockSpec((1, nb, _DIM), lambda b: (b, 0, 0)),
        out_shape=jax.ShapeDtypeStruct((bh, nb, _DIM), jnp.float32),
        compiler_params=pltpu.CompilerParams(
            dimension_semantics=("arbitrary",),
        ),
    )(q4)

    out = pl.pallas_call(
        _tail_body,
        in_specs=[
            pl.BlockSpec(memory_space=pltpu.SMEM),
            pl.BlockSpec((bh, nb, _DIM), lambda: (0, 0, 0)),
            pl.BlockSpec((nb, bh * _DIM), lambda: (0, 0)),
            pl.BlockSpec((nb, bh * _DIM), lambda: (0, 0)),
        ],
        out_specs=pl.BlockSpec((bh, nb, _DIM), lambda: (0, 0, 0)),
        out_shape=jax.ShapeDtypeStruct((bh, nb, _DIM), jnp.float32),
    )(scale, sqs, bk2, ws2)
    return out[:, :, : nb + 1]


# speedup vs baseline: 1.6103x; 1.0522x over previous
import jax
import jax.numpy as jnp
import numpy as np
from jax import lax
from jax.experimental import pallas as pl
from jax.experimental.pallas import tpu as pltpu
from jax.experimental.pallas import tpu_sc as plsc

_DIM = 128
_BUCKET = 64
_LANES = 16
_NBUF = 4
_NEG = -3.4028234663852886e38


def _sc_body(k_hbm, w_hbm, bk_hbm, ws_hbm, wbuf, kb0, kb1, kb2, kb3,
             obk, ows, sem0, sem1, sem2, sem3):
    bh = k_hbm.shape[0]
    ngrp = _DIM // _LANES
    wid = lax.axis_index("s") * 2 + lax.axis_index("c")
    pd = pl.ds(2 * wid, 2)

    pltpu.sync_copy(w_hbm.at[pd], wbuf)

    bufs = (kb0, kb1, kb2, kb3)
    sems = (sem0, sem1, sem2, sem3)
    zero = jnp.zeros((_LANES,), jnp.float32)

    def accumulate(kb, bh_idx):
        for jj in range(2):
            def s_body(u, accs):
                new = list(accs)
                for v in range(2):
                    s = 2 * u + v
                    wv = wbuf[jj, s]
                    for g in range(ngrp):
                        row = kb[jj, s, pl.ds(_LANES * g, _LANES)]
                        new[g] = new[g] + row
                        new[ngrp + g] = new[ngrp + g] + wv * row
                return tuple(new)

            accs = lax.fori_loop(0, _BUCKET // 2, s_body, (zero,) * (2 * ngrp))
            base = bh_idx * _DIM
            for g in range(ngrp):
                obk[jj, pl.ds(base + _LANES * g, _LANES)] = accs[g]
                ows[jj, pl.ds(base + _LANES * g, _LANES)] = accs[ngrp + g]

    for c in range(_NBUF):
        pltpu.async_copy(k_hbm.at[c, pd], bufs[c], sems[c])

    def t_body(t, carry):
        for b in range(_NBUF):
            c = _NBUF * t + b
            pltpu.make_async_copy(k_hbm.at[c, pd], bufs[b], sems[b]).wait()
            accumulate(bufs[b], c)

            @pl.when(c + _NBUF < bh)
            def _():
                pltpu.async_copy(k_hbm.at[c + _NBUF, pd], bufs[b], sems[b])

        return carry

    lax.fori_loop(0, bh // _NBUF, t_body, 0)

    pltpu.sync_copy(obk, bk_hbm.at[pd])
    pltpu.sync_copy(ows, ws_hbm.at[pd])


def _k_side_sc(k4, wtab):
    bh, nb = k4.shape[0], k4.shape[1]
    out_t = jax.ShapeDtypeStruct((nb, bh * _DIM), jnp.float32)
    kbuf = pltpu.VMEM((2, _BUCKET, _DIM), jnp.float32)
    f = pl.kernel(
        _sc_body,
        out_type=(out_t, out_t),
        mesh=plsc.VectorSubcoreMesh(core_axis_name="c", subcore_axis_name="s",
                                    num_cores=2, num_subcores=16),
        scratch_types=[
            pltpu.VMEM((2, _BUCKET, _LANES), jnp.float32),
            kbuf, kbuf, kbuf, kbuf,
            pltpu.VMEM((2, bh * _DIM), jnp.float32),
            pltpu.VMEM((2, bh * _DIM), jnp.float32),
            pltpu.SemaphoreType.DMA,
            pltpu.SemaphoreType.DMA,
            pltpu.SemaphoreType.DMA,
            pltpu.SemaphoreType.DMA,
        ],
    )
    return f(k4, wtab)


def _qside_body(q_ref, sq_ref):
    q3 = q_ref[0]
    nb = q3.shape[0]
    f32 = jnp.float32

    r64 = lax.broadcasted_iota(jnp.int32, (nb, _BUCKET), 0)
    c64 = lax.broadcasted_iota(jnp.int32, (nb, _BUCKET), 1)

    bq = jnp.sum(q3, axis=1)
    l_strict = (r64 > c64).astype(f32)
    pq = jnp.dot(l_strict, bq, preferred_element_type=f32,
                 precision=lax.Precision.HIGHEST)
    inv_cnt = 1.0 / (_BUCKET * r64[:, :1] + 1).astype(f32)
    sq_ref[0] = (pq + q3[:, 0, :]) * inv_cnt


def _tail_body(scale_ref, sq_ref, bk_ref, ws_ref, o_ref):
    bh = sq_ref.shape[0]
    nb = sq_ref.shape[1]
    f32 = jnp.float32
    scale = scale_ref[0]

    r64 = lax.broadcasted_iota(jnp.int32, (nb, _BUCKET), 0)
    c64 = lax.broadcasted_iota(jnp.int32, (nb, _BUCKET), 1)

    rinv = 1.0 / (_BUCKET * r64 + c64 + 1).astype(f32)
    h = jnp.sum(rinv, axis=1, keepdims=True)

    l_strict = (r64 > c64).astype(f32)
    pk2 = jnp.dot(l_strict, bk_ref[...], preferred_element_type=f32,
                  precision=lax.Precision.HIGHEST)
    sk2 = pk2 * h + ws_ref[...]

    ccol = lax.broadcasted_iota(jnp.int32, (nb, _DIM), 1)
    for b in range(bh):
        sq = sq_ref[b]
        sk = sk2[:, b * _DIM:(b + 1) * _DIM]
        r_core = lax.dot_general(sq, sk, (((1,), (1,)), ((), ())),
                                 preferred_element_type=f32) * scale
        r_core = jnp.where(c64 >= r64, _NEG, r_core)

        m = jnp.maximum(jnp.max(r_core, axis=1, keepdims=True), 0.0)
        e = jnp.exp(r_core - m)
        p0 = jnp.exp(-m)
        s = p0 + jnp.sum(e, axis=1, keepdims=True)
        p_core = e / s
        p0 = p0 / s
        v = jnp.maximum(jnp.max(p_core, axis=1, keepdims=True), p0)
        cand = jnp.where(p_core == v, c64 + 1, 2 * _BUCKET)
        amin = jnp.min(cand, axis=1, keepdims=True)
        amin = jnp.where(p0 == v, 0, amin)

        o_ref[b] = jnp.where(ccol == amin, v, 0.0)


def kernel(q, k, topk):
    bh, seq, dim = q.shape
    nb = seq // _BUCKET
    q4 = q.reshape(bh, nb, _BUCKET, dim)
    k4 = k.reshape(bh, nb, _BUCKET, dim)
    scale = (jnp.asarray(topk, jnp.float32) * (dim ** -0.5)).reshape(1)

    pos = np.arange(seq, dtype=np.float32).reshape(nb, _BUCKET)
    rinv = 1.0 / (pos + 1.0)
    w = np.cumsum(rinv[:, ::-1], axis=1)[:, ::-1]
    wtab = jnp.asarray(
        np.ascontiguousarray(
            np.broadcast_to(w[:, :, None], (nb, _BUCKET, _LANES))))

    bk2, ws2 = _k_side_sc(k4, wtab)

    sqs = pl.pallas_call(
        _qside_body,
        grid=(bh,),
        in_specs=[
            pl.BlockSpec((1, nb, _BUCKET, dim), lambda b: (b, 0, 0, 0)),
        ],
        out_specs=pl.BlockSpec((1, nb, _DIM), lambda b: (b, 0, 0)),
        out_shape=jax.ShapeDtypeStruct((bh, nb, _DIM), jnp.float32),
        compiler_params=pltpu.CompilerParams(
            dimension_semantics=("arbitrary",),
        ),
    )(q4)

    out = pl.pallas_call(
        _tail_body,
        in_specs=[
            pl.BlockSpec(memory_space=pltpu.SMEM),
            pl.BlockSpec((bh, nb, _DIM), lambda: (0, 0, 0)),
            pl.BlockSpec((nb, bh * _DIM), lambda: (0, 0)),
            pl.BlockSpec((nb, bh * _DIM), lambda: (0, 0)),
        ],
        out_specs=pl.BlockSpec((bh, nb, _DIM), lambda: (0, 0, 0)),
        out_shape=jax.ShapeDtypeStruct((bh, nb, _DIM), jnp.float32),
    )(scale, sqs, bk2, ws2)
    return out[:, :, : nb + 1]


# speedup vs baseline: 2.5096x; 1.5585x over previous
import jax
import jax.numpy as jnp
from jax import lax
from jax.experimental import pallas as pl
from jax.experimental.pallas import tpu as pltpu

_DIM = 128
_BUCKET = 64
_GROUP = 4
_NEG = -3.4028234663852886e38


def _body(scale_ref, q_ref, k_ref, o_ref):
    nb = q_ref.shape[1]
    f32 = jnp.float32

    r64 = lax.broadcasted_iota(jnp.int32, (nb, _BUCKET), 0)
    c64 = lax.broadcasted_iota(jnp.int32, (nb, _BUCKET), 1)

    rinv = 1.0 / (_BUCKET * r64 + c64 + 1).astype(f32)
    m_ge = (r64 >= c64).astype(f32)
    w = jnp.dot(rinv, m_ge, preferred_element_type=f32,
                precision=lax.Precision.HIGHEST)
    h = jnp.sum(rinv, axis=1, keepdims=True)
    l_strict = (r64 > c64).astype(f32)
    inv_cnt = 1.0 / (_BUCKET * r64[:, :1] + 1).astype(f32)
    ccol = lax.broadcasted_iota(jnp.int32, (nb, _DIM), 1)
    scale = scale_ref[0]

    for b in range(_GROUP):
        q3 = q_ref[b]
        k3 = k_ref[b]

        bq = jnp.sum(q3, axis=1)
        bk = jnp.sum(k3, axis=1)
        pq = jnp.dot(l_strict, bq, preferred_element_type=f32,
                     precision=lax.Precision.HIGHEST)
        pk = jnp.dot(l_strict, bk, preferred_element_type=f32,
                     precision=lax.Precision.HIGHEST)

        ws = jnp.sum(k3 * w[:, :, None], axis=1)
        sk = pk * h + ws
        sq = (pq + q3[:, 0, :]) * inv_cnt

        r_core = lax.dot_general(sq, sk, (((1,), (1,)), ((), ())),
                                 preferred_element_type=f32) * scale
        r_core = jnp.where(c64 >= r64, _NEG, r_core)

        m = jnp.maximum(jnp.max(r_core, axis=1, keepdims=True), 0.0)
        e = jnp.exp(r_core - m)
        p0 = jnp.exp(-m)
        s = p0 + jnp.sum(e, axis=1, keepdims=True)
        p_core = e / s
        p0 = p0 / s
        v = jnp.maximum(jnp.max(p_core, axis=1, keepdims=True), p0)
        cand = jnp.where(p_core == v, c64 + 1, 2 * _BUCKET)
        amin = jnp.min(cand, axis=1, keepdims=True)
        amin = jnp.where(p0 == v, 0, amin)

        o_ref[b] = jnp.where(ccol == amin, v, 0.0)


def kernel(q, k, topk):
    bh, seq, dim = q.shape
    nb = seq // _BUCKET
    q4 = q.reshape(bh, nb, _BUCKET, dim)
    k4 = k.reshape(bh, nb, _BUCKET, dim)
    scale = (jnp.asarray(topk, jnp.float32) * (dim ** -0.5)).reshape(1)

    out = pl.pallas_call(
        _body,
        grid=(bh // _GROUP,),
        in_specs=[
            pl.BlockSpec(memory_space=pltpu.SMEM),
            pl.BlockSpec((_GROUP, nb, _BUCKET, dim), lambda b: (b, 0, 0, 0)),
            pl.BlockSpec((_GROUP, nb, _BUCKET, dim), lambda b: (b, 0, 0, 0)),
        ],
        out_specs=pl.BlockSpec((_GROUP, nb, _DIM), lambda b: (b, 0, 0)),
        out_shape=jax.ShapeDtypeStruct((bh, nb, _DIM), jnp.float32),
        compiler_params=pltpu.CompilerParams(
            dimension_semantics=("arbitrary",),
        ),
    )(scale, q4, k4)
    return out[:, :, : nb + 1]


# speedup vs baseline: 2.5113x; 1.0007x over previous
import jax
import jax.numpy as jnp
from jax import lax
from jax.experimental import pallas as pl
from jax.experimental.pallas import tpu as pltpu

_DIM = 128
_BUCKET = 64
_GROUP = 4
_NEG = -3.4028234663852886e38


def _body(scale_ref, q_ref, k_ref, o_ref):
    nb = q_ref.shape[1]
    f32 = jnp.float32

    r64 = lax.broadcasted_iota(jnp.int32, (nb, _BUCKET), 0)
    c64 = lax.broadcasted_iota(jnp.int32, (nb, _BUCKET), 1)

    rinv = 1.0 / (_BUCKET * r64 + c64 + 1).astype(f32)
    m_ge = (r64 >= c64).astype(f32)
    w = jnp.dot(rinv, m_ge, preferred_element_type=f32,
                precision=lax.Precision.HIGHEST)
    h = jnp.sum(rinv, axis=1, keepdims=True)
    l_strict = (r64 > c64).astype(f32)
    inv_cnt = 1.0 / (_BUCKET * r64[:, :1] + 1).astype(f32)
    ccol = lax.broadcasted_iota(jnp.int32, (nb, _DIM), 1)
    scale = scale_ref[0]

    for b in range(_GROUP):
        q3 = q_ref[b]
        k3 = k_ref[b]

        bq = jnp.sum(q3, axis=1)
        bk = jnp.sum(k3, axis=1)
        pq = jnp.dot(l_strict, bq, preferred_element_type=f32,
                     precision=lax.Precision.HIGHEST)
        pk = jnp.dot(l_strict, bk, preferred_element_type=f32,
                     precision=lax.Precision.HIGHEST)

        ws = jnp.sum(k3 * w[:, :, None], axis=1)
        sk = pk * h + ws
        sq = (pq + q3[:, 0, :]) * inv_cnt

        r_core = lax.dot_general(sq, sk, (((1,), (1,)), ((), ())),
                                 preferred_element_type=f32) * scale
        r_core = jnp.where(c64 >= r64, _NEG, r_core)

        m = jnp.maximum(jnp.max(r_core, axis=1, keepdims=True), 0.0)
        e = jnp.exp(r_core - m)
        p0 = jnp.exp(-m)
        s = p0 + jnp.sum(e, axis=1, keepdims=True)
        p_core = e / s
        p0 = p0 / s
        v = jnp.maximum(jnp.max(p_core, axis=1, keepdims=True), p0)
        cand = jnp.where(p_core == v, c64 + 1, 2 * _BUCKET)
        amin = jnp.min(cand, axis=1, keepdims=True)
        amin = jnp.where(p0 == v, 0, amin)

        o_ref[b] = jnp.where(ccol == amin, v, 0.0)[:, : nb + 1]


def kernel(q, k, topk):
    bh, seq, dim = q.shape
    nb = seq // _BUCKET
    q4 = q.reshape(bh, nb, _BUCKET, dim)
    k4 = k.reshape(bh, nb, _BUCKET, dim)
    scale = (jnp.asarray(topk, jnp.float32) * (dim ** -0.5)).reshape(1)

    out = pl.pallas_call(
        _body,
        grid=(bh // _GROUP,),
        in_specs=[
            pl.BlockSpec(memory_space=pltpu.SMEM),
            pl.BlockSpec((_GROUP, nb, _BUCKET, dim), lambda b: (b, 0, 0, 0)),
            pl.BlockSpec((_GROUP, nb, _BUCKET, dim), lambda b: (b, 0, 0, 0)),
        ],
        out_specs=pl.BlockSpec((_GROUP, nb, nb + 1), lambda b: (b, 0, 0)),
        out_shape=jax.ShapeDtypeStruct((bh, nb, nb + 1), jnp.float32),
        compiler_params=pltpu.CompilerParams(
            dimension_semantics=("arbitrary",),
        ),
    )(scale, q4, k4)
    return out
